# baseline (speedup 1.0000x reference)
import functools

import jax
import jax.numpy as jnp
from jax import lax
from jax.experimental import pallas as pl
from jax.experimental.pallas import tpu as pltpu
from jax.experimental.pallas import tpu_sc as plsc

N = 10000
E = 320000
D = 128
B = 64

NC = 2
NS = 16
NW = NC * NS
C = 64
KPH = 56
KCH = 168
KBUF = 64
KDATA = 176
NPAD = 10112
ROWS_PER_TILE = NPAD // NS


def _conv_body(h_hbm, src_hbm, dst_hbm, w_hbm, zero_hbm, out_hbm,
               srcb, dstb, wb, rows0, rows1, acc, gsem0, gsem1):
    c = lax.axis_index("c")
    s = lax.axis_index("s")
    wid = c * NS + s

    r0 = s * ROWS_PER_TILE
    pltpu.sync_copy(zero_hbm.at[pl.ds(r0, ROWS_PER_TILE)],
                    acc.at[pl.ds(r0, ROWS_PER_TILE)])

    def stage(phase0):
        pltpu.sync_copy(src_hbm.at[wid, pl.ds(phase0, KBUF)], srcb)
        pltpu.sync_copy(dst_hbm.at[wid, pl.ds(phase0, KBUF)], dstb)
        pltpu.sync_copy(w_hbm.at[wid, pl.ds(phase0, KBUF)], wb)

    def scale(rows, j):
        def grp_body(i16, carry2):
            w16 = wb[j, pl.ds(i16 * 16, 16)]
            for l in range(16):
                wl = w16[l]
                row = i16 * 16 + l
                for jj in range(D // 16):
                    sl = pl.ds(jj * 16, 16)
                    rows[row, sl] = rows[row, sl] * wl
            return carry2
        lax.fori_loop(0, C // 16, grp_body, 0)

    def phase():
        pltpu.async_copy(h_hbm.at[srcb.at[0]], rows0, gsem0)

        def pair(i, carry):
            j = 2 * i
            pltpu.make_async_copy(h_hbm.at[srcb.at[j]], rows0, gsem0).wait()
            pltpu.async_copy(h_hbm.at[srcb.at[j + 1]], rows1, gsem1)
            scale(rows0, j)
            pltpu.sync_copy(rows0, acc.at[dstb.at[j]], add=True)

            pltpu.make_async_copy(h_hbm.at[srcb.at[j + 1]], rows1,
                                  gsem1).wait()
            pltpu.async_copy(h_hbm.at[srcb.at[j + 2]], rows0, gsem0)
            scale(rows1, j + 1)
            pltpu.sync_copy(rows1, acc.at[dstb.at[j + 1]], add=True)
            return carry
        lax.fori_loop(0, KPH // 2, pair, 0)
        pltpu.make_async_copy(h_hbm.at[srcb.at[KPH]], rows0, gsem0).wait()

    stage(0)
    plsc.subcore_barrier()
    phase()
    stage(KPH)
    phase()
    stage(2 * KPH)
    phase()

    plsc.subcore_barrier()
    pltpu.sync_copy(acc.at[pl.ds(r0, ROWS_PER_TILE)],
                    out_hbm.at[c, pl.ds(r0, ROWS_PER_TILE)])


_conv = pl.kernel(
    _conv_body,
    out_type=jax.ShapeDtypeStruct((NC, NPAD, D), jnp.float32),
    mesh=plsc.VectorSubcoreMesh(core_axis_name="c", subcore_axis_name="s",
                                num_cores=NC, num_subcores=NS),
    scratch_types=[
        pltpu.VMEM((KBUF, C), jnp.int32),
        pltpu.VMEM((KBUF, C), jnp.int32),
        pltpu.VMEM((KBUF, C), jnp.float32),
        pltpu.VMEM((C, D), jnp.float32),
        pltpu.VMEM((C, D), jnp.float32),
        pltpu.VMEM_SHARED((NPAD, D), jnp.float32),
        pltpu.SemaphoreType.DMA,
        pltpu.SemaphoreType.DMA,
    ],
)


def _mm_body(x_ref, w_ref, o_ref):
    o_ref[...] = jnp.dot(x_ref[...], w_ref[...],
                         preferred_element_type=jnp.float32)


def _mm(x, w):
    return pl.pallas_call(
        _mm_body,
        out_shape=jax.ShapeDtypeStruct((x.shape[0], w.shape[1]), jnp.float32),
    )(x, w)


def _mid_body(p_ref, b_ref, w_ref, o_ref):
    x1 = jnp.maximum(p_ref[0, :N] + p_ref[1, :N] + b_ref[...], 0.0)
    o_ref[...] = jnp.dot(x1, w_ref[...], preferred_element_type=jnp.float32)


def _mid(p, b, w):
    return pl.pallas_call(
        _mid_body,
        out_shape=jax.ShapeDtypeStruct((N, D), jnp.float32),
    )(p, b.reshape(1, D), w)


def _head_body(p_ref, b2_ref, batch_ref, wg1_ref, bg1_ref, wg2_ref, bg2_ref,
               wl1_ref, bl1_ref, wl2_ref, bl2_ref, o_ref):
    x2 = p_ref[0, :N] + p_ref[1, :N] + b2_ref[...]
    t = jnp.maximum(jnp.dot(x2, wg1_ref[...],
                            preferred_element_type=jnp.float32)
                    + bg1_ref[...], 0.0)
    g = jnp.dot(t, wg2_ref[...], preferred_element_type=jnp.float32) \
        + bg2_ref[...]

    gid = lax.broadcasted_iota(jnp.int32, (N, B), 1)
    onehot = batch_ref[...] == gid
    onehotf = onehot.astype(jnp.float32)

    m = jnp.max(jnp.where(onehot, g, -1e30), axis=0, keepdims=True)
    m_node = jnp.sum(onehotf * m, axis=1, keepdims=True)
    e = jnp.exp(g - m_node)
    denom = jnp.sum(onehotf * e, axis=0, keepdims=True)
    denom_node = jnp.sum(onehotf * denom, axis=1, keepdims=True)
    alpha = e / (denom_node + 1e-16)

    pooled = lax.dot_general(onehotf, alpha * x2, (((0,), (0,)), ((), ())),
                             preferred_element_type=jnp.float32)
    h = jnp.maximum(jnp.dot(pooled, wl1_ref[...],
                            preferred_element_type=jnp.float32)
                    + bl1_ref[...], 0.0)
    o_ref[...] = jnp.dot(h, wl2_ref[...],
                         preferred_element_type=jnp.float32) + bl2_ref[...]


def _head(p, b2, batch2d, Wg1, bg1, Wg2, bg2, Wl1, bl1, Wl2, bl2):
    return pl.pallas_call(
        _head_body,
        out_shape=jax.ShapeDtypeStruct((B, 1), jnp.float32),
    )(p, b2.reshape(1, D), batch2d, Wg1, bg1.reshape(1, D), Wg2,
      bg2.reshape(1, 1), Wl1, bl1.reshape(1, D), Wl2, bl2.reshape(1, 1))


def kernel(x, edge_index, edge_attr, batch, W1, b1, W2, b2,
           Wg1, bg1, Wg2, bg2, Wl1, bl1, Wl2, bl2):
    pad = NW * KCH * C - E

    def lay(a):
        a3 = jnp.pad(a, (0, pad)).reshape(NW, KCH, C)
        return jnp.pad(a3, ((0, 0), (0, KDATA - KCH), (0, 0)))
    src3 = lay(edge_index[0])
    dst3 = lay(edge_index[1])
    w3 = lay(edge_attr)
    zeros_nd = jnp.zeros((NPAD, D), jnp.float32)

    h1 = _mm(x, W1)
    p1 = _conv(h1, src3, dst3, w3, zeros_nd)
    h2 = _mid(p1, b1, W2)
    p2 = _conv(h2, src3, dst3, w3, zeros_nd)
    out = _head(p2, b2, batch.reshape(N, 1), Wg1, bg1, Wg2, bg2,
                Wl1, bl1, Wl2, bl2)
    return out[:, 0]


# speedup vs baseline: 1.7864x; 1.7864x over previous
import functools

import jax
import jax.numpy as jnp
from jax import lax
from jax.experimental import pallas as pl
from jax.experimental.pallas import tpu as pltpu
from jax.experimental.pallas import tpu_sc as plsc

N = 10000
E = 320000
D = 128
B = 64

NC = 2
NS = 16
NW = NC * NS
C = 128
KPH = 40
KCH = 80
KBUF = 40
KDATA = 80
NPAD = 10112
ROWS_PER_TILE = NPAD // NS


def _conv_body(h_hbm, src_hbm, dst_hbm, w_hbm, zero_hbm, out_hbm,
               srcb, dstb, wb, rows0, rows1, acc, gsem0, gsem1):
    c = lax.axis_index("c")
    s = lax.axis_index("s")
    wid = c * NS + s

    r0 = s * ROWS_PER_TILE
    pltpu.sync_copy(zero_hbm.at[pl.ds(r0, ROWS_PER_TILE)],
                    acc.at[pl.ds(r0, ROWS_PER_TILE)])

    def stage(phase0):
        pltpu.sync_copy(src_hbm.at[wid, pl.ds(phase0, KBUF)], srcb)
        pltpu.sync_copy(dst_hbm.at[wid, pl.ds(phase0, KBUF)], dstb)
        pltpu.sync_copy(w_hbm.at[wid, pl.ds(phase0, KBUF)], wb)

    def scale(rows, j):
        def grp_body(i16, carry2):
            w16 = wb[j, pl.ds(i16 * 16, 16)]
            for l in range(16):
                wl = w16[l]
                row = i16 * 16 + l
                for jj in range(D // 16):
                    sl = pl.ds(jj * 16, 16)
                    rows[row, sl] = rows[row, sl] * wl
            return carry2
        lax.fori_loop(0, C // 16, grp_body, 0)

    def phase():
        def body(j, carry):
            pltpu.async_copy(h_hbm.at[srcb.at[j]], rows0, gsem0).wait()
            scale(rows0, j)
            pltpu.sync_copy(rows0, acc.at[dstb.at[j]], add=True)
            return carry
        lax.fori_loop(0, KPH, body, 0)

    stage(0)
    plsc.subcore_barrier()
    phase()
    stage(KPH)
    phase()

    plsc.subcore_barrier()
    pltpu.sync_copy(acc.at[pl.ds(r0, ROWS_PER_TILE)],
                    out_hbm.at[c, pl.ds(r0, ROWS_PER_TILE)])


_conv = pl.kernel(
    _conv_body,
    out_type=jax.ShapeDtypeStruct((NC, NPAD, D), jnp.float32),
    mesh=plsc.VectorSubcoreMesh(core_axis_name="c", subcore_axis_name="s",
                                num_cores=NC, num_subcores=NS),
    scratch_types=[
        pltpu.VMEM((KBUF, C), jnp.int32),
        pltpu.VMEM((KBUF, C), jnp.int32),
        pltpu.VMEM((KBUF, C), jnp.float32),
        pltpu.VMEM((C, D), jnp.float32),
        pltpu.VMEM((C, D), jnp.float32),
        pltpu.VMEM_SHARED((NPAD, D), jnp.float32),
        pltpu.SemaphoreType.DMA,
        pltpu.SemaphoreType.DMA,
    ],
)


def _mm_body(x_ref, w_ref, o_ref):
    o_ref[...] = jnp.dot(x_ref[...], w_ref[...],
                         preferred_element_type=jnp.float32)


def _mm(x, w):
    return pl.pallas_call(
        _mm_body,
        out_shape=jax.ShapeDtypeStruct((x.shape[0], w.shape[1]), jnp.float32),
    )(x, w)


def _mid_body(p_ref, b_ref, w_ref, o_ref):
    x1 = jnp.maximum(p_ref[0, :N] + p_ref[1, :N] + b_ref[...], 0.0)
    o_ref[...] = jnp.dot(x1, w_ref[...], preferred_element_type=jnp.float32)


def _mid(p, b, w):
    return pl.pallas_call(
        _mid_body,
        out_shape=jax.ShapeDtypeStruct((N, D), jnp.float32),
    )(p, b.reshape(1, D), w)


def _head_body(p_ref, b2_ref, batch_ref, wg1_ref, bg1_ref, wg2_ref, bg2_ref,
               wl1_ref, bl1_ref, wl2_ref, bl2_ref, o_ref):
    x2 = p_ref[0, :N] + p_ref[1, :N] + b2_ref[...]
    t = jnp.maximum(jnp.dot(x2, wg1_ref[...],
                            preferred_element_type=jnp.float32)
                    + bg1_ref[...], 0.0)
    g = jnp.dot(t, wg2_ref[...], preferred_element_type=jnp.float32) \
        + bg2_ref[...]

    gid = lax.broadcasted_iota(jnp.int32, (N, B), 1)
    onehot = batch_ref[...] == gid
    onehotf = onehot.astype(jnp.float32)

    m = jnp.max(jnp.where(onehot, g, -1e30), axis=0, keepdims=True)
    m_node = jnp.sum(onehotf * m, axis=1, keepdims=True)
    e = jnp.exp(g - m_node)
    denom = jnp.sum(onehotf * e, axis=0, keepdims=True)
    denom_node = jnp.sum(onehotf * denom, axis=1, keepdims=True)
    alpha = e / (denom_node + 1e-16)

    pooled = lax.dot_general(onehotf, alpha * x2, (((0,), (0,)), ((), ())),
                             preferred_element_type=jnp.float32)
    h = jnp.maximum(jnp.dot(pooled, wl1_ref[...],
                            preferred_element_type=jnp.float32)
                    + bl1_ref[...], 0.0)
    o_ref[...] = jnp.dot(h, wl2_ref[...],
                         preferred_element_type=jnp.float32) + bl2_ref[...]


def _head(p, b2, batch2d, Wg1, bg1, Wg2, bg2, Wl1, bl1, Wl2, bl2):
    return pl.pallas_call(
        _head_body,
        out_shape=jax.ShapeDtypeStruct((B, 1), jnp.float32),
    )(p, b2.reshape(1, D), batch2d, Wg1, bg1.reshape(1, D), Wg2,
      bg2.reshape(1, 1), Wl1, bl1.reshape(1, D), Wl2, bl2.reshape(1, 1))


def kernel(x, edge_index, edge_attr, batch, W1, b1, W2, b2,
           Wg1, bg1, Wg2, bg2, Wl1, bl1, Wl2, bl2):
    pad = NW * KCH * C - E

    def lay(a):
        a3 = jnp.pad(a, (0, pad)).reshape(NW, KCH, C)
        return jnp.pad(a3, ((0, 0), (0, KDATA - KCH), (0, 0)))
    src3 = lay(edge_index[0])
    dst3 = lay(edge_index[1])
    w3 = lay(edge_attr)
    zeros_nd = jnp.zeros((NPAD, D), jnp.float32)

    h1 = _mm(x, W1)
    p1 = _conv(h1, src3, dst3, w3, zeros_nd)
    h2 = _mid(p1, b1, W2)
    p2 = _conv(h2, src3, dst3, w3, zeros_nd)
    out = _head(p2, b2, batch.reshape(N, 1), Wg1, bg1, Wg2, bg2,
                Wl1, bl1, Wl2, bl2)
    return out[:, 0]


# speedup vs baseline: 1.7964x; 1.0056x over previous
import functools

import jax
import jax.numpy as jnp
from jax import lax
from jax.experimental import pallas as pl
from jax.experimental.pallas import tpu as pltpu
from jax.experimental.pallas import tpu_sc as plsc

N = 10000
E = 320000
D = 128
B = 64

NC = 2
NS = 16
NW = NC * NS
C = 128
KPH = 80
KCH = 80
KBUF = 80
KDATA = 80
NPAD = 10112
ROWS_PER_TILE = NPAD // NS


def _conv_body(h_hbm, src_hbm, dst_hbm, w_hbm, zero_hbm, out_hbm,
               srcb, dstb, wb, rows0, acc, gsem0):
    c = lax.axis_index("c")
    s = lax.axis_index("s")
    wid = c * NS + s

    r0 = s * ROWS_PER_TILE
    pltpu.sync_copy(zero_hbm.at[pl.ds(r0, ROWS_PER_TILE)],
                    acc.at[pl.ds(r0, ROWS_PER_TILE)])

    def stage(phase0):
        pltpu.sync_copy(src_hbm.at[wid, pl.ds(phase0, KBUF)], srcb)
        pltpu.sync_copy(dst_hbm.at[wid, pl.ds(phase0, KBUF)], dstb)
        pltpu.sync_copy(w_hbm.at[wid, pl.ds(phase0, KBUF)], wb)

    def scale(rows, j):
        def grp_body(i16, carry2):
            w16 = wb[j, pl.ds(i16 * 16, 16)]
            for l in range(16):
                wl = w16[l]
                row = i16 * 16 + l
                for jj in range(D // 16):
                    sl = pl.ds(jj * 16, 16)
                    rows[row, sl] = rows[row, sl] * wl
            return carry2
        lax.fori_loop(0, C // 16, grp_body, 0)

    def phase():
        def body(j, carry):
            pltpu.async_copy(h_hbm.at[srcb.at[j]], rows0, gsem0).wait()
            scale(rows0, j)
            pltpu.sync_copy(rows0, acc.at[dstb.at[j]], add=True)
            return carry
        lax.fori_loop(0, KPH, body, 0)

    stage(0)
    plsc.subcore_barrier()
    phase()

    plsc.subcore_barrier()
    pltpu.sync_copy(acc.at[pl.ds(r0, ROWS_PER_TILE)],
                    out_hbm.at[c, pl.ds(r0, ROWS_PER_TILE)])


_conv = pl.kernel(
    _conv_body,
    out_type=jax.ShapeDtypeStruct((NC, NPAD, D), jnp.float32),
    mesh=plsc.VectorSubcoreMesh(core_axis_name="c", subcore_axis_name="s",
                                num_cores=NC, num_subcores=NS),
    scratch_types=[
        pltpu.VMEM((KBUF, C), jnp.int32),
        pltpu.VMEM((KBUF, C), jnp.int32),
        pltpu.VMEM((KBUF, C), jnp.float32),
        pltpu.VMEM((C, D), jnp.float32),
        pltpu.VMEM_SHARED((NPAD, D), jnp.float32),
        pltpu.SemaphoreType.DMA,
    ],
)


def _mm_body(x_ref, w_ref, o_ref):
    o_ref[...] = jnp.dot(x_ref[...], w_ref[...],
                         preferred_element_type=jnp.float32)


def _mm(x, w):
    return pl.pallas_call(
        _mm_body,
        out_shape=jax.ShapeDtypeStruct((x.shape[0], w.shape[1]), jnp.float32),
    )(x, w)


def _mid_body(p_ref, b_ref, w_ref, o_ref):
    x1 = jnp.maximum(p_ref[0, :N] + p_ref[1, :N] + b_ref[...], 0.0)
    o_ref[...] = jnp.dot(x1, w_ref[...], preferred_element_type=jnp.float32)


def _mid(p, b, w):
    return pl.pallas_call(
        _mid_body,
        out_shape=jax.ShapeDtypeStruct((N, D), jnp.float32),
    )(p, b.reshape(1, D), w)


def _head_body(p_ref, b2_ref, batch_ref, wg1_ref, bg1_ref, wg2_ref, bg2_ref,
               wl1_ref, bl1_ref, wl2_ref, bl2_ref, o_ref):
    x2 = p_ref[0, :N] + p_ref[1, :N] + b2_ref[...]
    t = jnp.maximum(jnp.dot(x2, wg1_ref[...],
                            preferred_element_type=jnp.float32)
                    + bg1_ref[...], 0.0)
    g = jnp.dot(t, wg2_ref[...], preferred_element_type=jnp.float32) \
        + bg2_ref[...]

    gid = lax.broadcasted_iota(jnp.int32, (N, B), 1)
    onehot = batch_ref[...] == gid
    onehotf = onehot.astype(jnp.float32)

    m = jnp.max(jnp.where(onehot, g, -1e30), axis=0, keepdims=True)
    m_node = jnp.sum(onehotf * m, axis=1, keepdims=True)
    e = jnp.exp(g - m_node)
    denom = jnp.sum(onehotf * e, axis=0, keepdims=True)
    denom_node = jnp.sum(onehotf * denom, axis=1, keepdims=True)
    alpha = e / (denom_node + 1e-16)

    pooled = lax.dot_general(onehotf, alpha * x2, (((0,), (0,)), ((), ())),
                             preferred_element_type=jnp.float32)
    h = jnp.maximum(jnp.dot(pooled, wl1_ref[...],
                            preferred_element_type=jnp.float32)
                    + bl1_ref[...], 0.0)
    o_ref[...] = jnp.dot(h, wl2_ref[...],
                         preferred_element_type=jnp.float32) + bl2_ref[...]


def _head(p, b2, batch2d, Wg1, bg1, Wg2, bg2, Wl1, bl1, Wl2, bl2):
    return pl.pallas_call(
        _head_body,
        out_shape=jax.ShapeDtypeStruct((B, 1), jnp.float32),
    )(p, b2.reshape(1, D), batch2d, Wg1, bg1.reshape(1, D), Wg2,
      bg2.reshape(1, 1), Wl1, bl1.reshape(1, D), Wl2, bl2.reshape(1, 1))


def kernel(x, edge_index, edge_attr, batch, W1, b1, W2, b2,
           Wg1, bg1, Wg2, bg2, Wl1, bl1, Wl2, bl2):
    pad = NW * KCH * C - E

    def lay(a):
        a3 = jnp.pad(a, (0, pad)).reshape(NW, KCH, C)
        return jnp.pad(a3, ((0, 0), (0, KDATA - KCH), (0, 0)))
    src3 = lay(edge_index[0])
    dst3 = lay(edge_index[1])
    w3 = lay(edge_attr)
    zeros_nd = jnp.zeros((NPAD, D), jnp.float32)

    h1 = _mm(x, W1)
    p1 = _conv(h1, src3, dst3, w3, zeros_nd)
    h2 = _mid(p1, b1, W2)
    p2 = _conv(h2, src3, dst3, w3, zeros_nd)
    out = _head(p2, b2, batch.reshape(N, 1), Wg1, bg1, Wg2, bg2,
                Wl1, bl1, Wl2, bl2)
    return out[:, 0]


# speedup vs baseline: 4.6934x; 2.6127x over previous
import functools

import jax
import jax.numpy as jnp
from jax import lax
from jax.experimental import pallas as pl
from jax.experimental.pallas import tpu as pltpu
from jax.experimental.pallas import tpu_sc as plsc

N = 10000
E = 320000
D = 128
B = 64

NC = 2
NS = 16
NW = NC * NS
C = 128
KPH = 80
KCH = 80
KBUF = 80
KDATA = 80
NPAD = 10112
ROWS_PER_TILE = NPAD // NS


def _conv_body(h_hbm, src_hbm, dst_hbm, w_hbm, zero_hbm, out_hbm,
               srcb, dstb, wb, rows0, acc, gsem0):
    c = lax.axis_index("c")
    s = lax.axis_index("s")
    wid = c * NS + s

    r0 = s * ROWS_PER_TILE
    pltpu.sync_copy(zero_hbm.at[pl.ds(r0, ROWS_PER_TILE)],
                    acc.at[pl.ds(r0, ROWS_PER_TILE)])

    def stage(phase0):
        pltpu.sync_copy(src_hbm.at[wid, pl.ds(phase0, KBUF)], srcb)
        pltpu.sync_copy(dst_hbm.at[wid, pl.ds(phase0, KBUF)], dstb)
        pltpu.sync_copy(w_hbm.at[wid, pl.ds(phase0, KBUF)], wb)

    def scale(rows, j):
        def grp_body(i16, carry2):
            w16 = wb[j, pl.ds(i16 * 16, 16)]
            for l in range(16):
                wl = w16[l]
                row = i16 * 16 + l
                for jj in range(D // 16):
                    sl = pl.ds(jj * 16, 16)
                    rows[row, sl] = rows[row, sl] * wl
            return carry2
        lax.fori_loop(0, C // 16, grp_body, 0)

    def phase():
        def body(j, carry):
            pltpu.async_copy(h_hbm.at[srcb.at[j]], rows0, gsem0).wait()
            scale(rows0, j)
            pltpu.sync_copy(rows0, acc.at[dstb.at[j]], add=True)
            return carry
        lax.fori_loop(0, KPH, body, 0)

    stage(0)
    plsc.subcore_barrier()
    phase()

    plsc.subcore_barrier()
    pltpu.sync_copy(acc.at[pl.ds(r0, ROWS_PER_TILE)],
                    out_hbm.at[c, pl.ds(r0, ROWS_PER_TILE)])


_conv = pl.kernel(
    _conv_body,
    out_type=jax.ShapeDtypeStruct((NC, NPAD, D), jnp.float32),
    mesh=plsc.VectorSubcoreMesh(core_axis_name="c", subcore_axis_name="s",
                                num_cores=NC, num_subcores=NS),
    scratch_types=[
        pltpu.VMEM((KBUF, C), jnp.int32),
        pltpu.VMEM((KBUF, C), jnp.int32),
        pltpu.VMEM((KBUF, C), jnp.float32),
        pltpu.VMEM((C, D), jnp.float32),
        pltpu.VMEM_SHARED((NPAD, D), jnp.float32),
        pltpu.SemaphoreType.DMA,
    ],
)


def _mm_body(x_ref, w_ref, o_ref):
    o_ref[...] = jnp.dot(x_ref[...], w_ref[...],
                         preferred_element_type=jnp.float32)


def _mm(x, w):
    return pl.pallas_call(
        _mm_body,
        out_shape=jax.ShapeDtypeStruct((x.shape[0], w.shape[1]), jnp.float32),
    )(x, w)


def _mid_body(p_ref, b_ref, w_ref, o_ref):
    x1 = jnp.maximum(p_ref[0, :N] + p_ref[1, :N] + b_ref[...], 0.0)
    o_ref[...] = jnp.dot(x1, w_ref[...], preferred_element_type=jnp.float32)


def _mid(p, b, w):
    return pl.pallas_call(
        _mid_body,
        out_shape=jax.ShapeDtypeStruct((N, D), jnp.float32),
    )(p, b.reshape(1, D), w)


def _head_body(p_ref, b2_ref, batch_ref, wg1_ref, bg1_ref, wg2_ref, bg2_ref,
               wl1_ref, bl1_ref, wl2_ref, bl2_ref, o_ref):
    x2 = p_ref[0, :N] + p_ref[1, :N] + b2_ref[...]
    t = jnp.maximum(jnp.dot(x2, wg1_ref[...],
                            preferred_element_type=jnp.float32)
                    + bg1_ref[...], 0.0)
    g = jnp.dot(t, wg2_ref[...], preferred_element_type=jnp.float32) \
        + bg2_ref[...]

    gid = lax.broadcasted_iota(jnp.int32, (N, B), 1)
    onehot = batch_ref[...] == gid
    onehotf = onehot.astype(jnp.float32)

    m = jnp.max(jnp.where(onehot, g, -1e30), axis=0, keepdims=True)
    m_node = jnp.sum(onehotf * m, axis=1, keepdims=True)
    e = jnp.exp(g - m_node)
    denom = jnp.sum(onehotf * e, axis=0, keepdims=True)
    denom_node = jnp.sum(onehotf * denom, axis=1, keepdims=True)
    alpha = e / (denom_node + 1e-16)

    pooled = lax.dot_general(onehotf, alpha * x2, (((0,), (0,)), ((), ())),
                             preferred_element_type=jnp.float32)
    h = jnp.maximum(jnp.dot(pooled, wl1_ref[...],
                            preferred_element_type=jnp.float32)
                    + bl1_ref[...], 0.0)
    o_ref[...] = jnp.dot(h, wl2_ref[...],
                         preferred_element_type=jnp.float32) + bl2_ref[...]


def _head(p, b2, batch2d, Wg1, bg1, Wg2, bg2, Wl1, bl1, Wl2, bl2):
    return pl.pallas_call(
        _head_body,
        out_shape=jax.ShapeDtypeStruct((B, 1), jnp.float32),
    )(p, b2.reshape(1, D), batch2d, Wg1, bg1.reshape(1, D), Wg2,
      bg2.reshape(1, 1), Wl1, bl1.reshape(1, D), Wl2, bl2.reshape(1, 1))


def kernel(x, edge_index, edge_attr, batch, W1, b1, W2, b2,
           Wg1, bg1, Wg2, bg2, Wl1, bl1, Wl2, bl2):
    pad = NW * KCH * C - E
    pad_idx = jnp.arange(pad, dtype=jnp.int32) % N

    def lay(a, p):
        return jnp.concatenate([a, p]).reshape(NW, KCH, C)
    src3 = lay(edge_index[0], pad_idx)
    dst3 = lay(edge_index[1], pad_idx)
    w3 = lay(edge_attr, jnp.zeros((pad,), jnp.float32))
    zeros_nd = jnp.zeros((NPAD, D), jnp.float32)

    h1 = _mm(x, W1)
    p1 = _conv(h1, src3, dst3, w3, zeros_nd)
    h2 = _mid(p1, b1, W2)
    p2 = _conv(h2, src3, dst3, w3, zeros_nd)
    out = _head(p2, b2, batch.reshape(N, 1), Wg1, bg1, Wg2, bg2,
                Wl1, bl1, Wl2, bl2)
    return out[:, 0]


# speedup vs baseline: 5.2109x; 1.1102x over previous
import functools

import jax
import jax.numpy as jnp
from jax import lax
from jax.experimental import pallas as pl
from jax.experimental.pallas import tpu as pltpu
from jax.experimental.pallas import tpu_sc as plsc

N = 10000
E = 320000
D = 128
B = 64

NC = 2
NS = 16
NW = NC * NS
C = 128
KPH = 40
KCH = 80
KBUF = 40
KDATA = 80
NPAD = 10112
ROWS_PER_TILE = NPAD // NS


def _conv_body(h_hbm, src_hbm, dst_hbm, w_hbm, zero_hbm, out_hbm,
               srcb, dstb, wb, rows0, rows1, acc, gsem0, gsem1):
    c = lax.axis_index("c")
    s = lax.axis_index("s")
    wid = c * NS + s

    r0 = s * ROWS_PER_TILE
    pltpu.sync_copy(zero_hbm.at[pl.ds(r0, ROWS_PER_TILE)],
                    acc.at[pl.ds(r0, ROWS_PER_TILE)])

    def stage(phase0):
        pltpu.sync_copy(src_hbm.at[wid, pl.ds(phase0, KBUF)], srcb)
        pltpu.sync_copy(dst_hbm.at[wid, pl.ds(phase0, KBUF)], dstb)
        pltpu.sync_copy(w_hbm.at[wid, pl.ds(phase0, KBUF)], wb)

    def scale(rows, j):
        def grp_body(i16, carry2):
            w16 = wb[j, pl.ds(i16 * 16, 16)]
            for l in range(16):
                wl = w16[l]
                row = i16 * 16 + l
                for jj in range(D // 16):
                    sl = pl.ds(jj * 16, 16)
                    rows[row, sl] = rows[row, sl] * wl
            return carry2
        lax.fori_loop(0, C // 16, grp_body, 0)

    def phase():
        def pair(i, carry):
            j = 2 * i
            d0 = pltpu.async_copy(h_hbm.at[srcb.at[j]], rows0, gsem0)
            d1 = pltpu.async_copy(h_hbm.at[srcb.at[j + 1]], rows1, gsem1)
            d0.wait()
            scale(rows0, j)
            pltpu.sync_copy(rows0, acc.at[dstb.at[j]], add=True)
            d1.wait()
            scale(rows1, j + 1)
            pltpu.sync_copy(rows1, acc.at[dstb.at[j + 1]], add=True)
            return carry
        lax.fori_loop(0, KPH // 2, pair, 0)

    stage(0)
    plsc.subcore_barrier()
    phase()
    stage(KPH)
    phase()

    plsc.subcore_barrier()
    pltpu.sync_copy(acc.at[pl.ds(r0, ROWS_PER_TILE)],
                    out_hbm.at[c, pl.ds(r0, ROWS_PER_TILE)])


_conv = pl.kernel(
    _conv_body,
    out_type=jax.ShapeDtypeStruct((NC, NPAD, D), jnp.float32),
    mesh=plsc.VectorSubcoreMesh(core_axis_name="c", subcore_axis_name="s",
                                num_cores=NC, num_subcores=NS),
    scratch_types=[
        pltpu.VMEM((KBUF, C), jnp.int32),
        pltpu.VMEM((KBUF, C), jnp.int32),
        pltpu.VMEM((KBUF, C), jnp.float32),
        pltpu.VMEM((C, D), jnp.float32),
        pltpu.VMEM((C, D), jnp.float32),
        pltpu.VMEM_SHARED((NPAD, D), jnp.float32),
        pltpu.SemaphoreType.DMA,
        pltpu.SemaphoreType.DMA,
    ],
)


def _mm_body(x_ref, w_ref, o_ref):
    o_ref[...] = jnp.dot(x_ref[...], w_ref[...],
                         preferred_element_type=jnp.float32)


def _mm(x, w):
    return pl.pallas_call(
        _mm_body,
        out_shape=jax.ShapeDtypeStruct((x.shape[0], w.shape[1]), jnp.float32),
    )(x, w)


def _mid_body(p_ref, b_ref, w_ref, o_ref):
    x1 = jnp.maximum(p_ref[0, :N] + p_ref[1, :N] + b_ref[...], 0.0)
    o_ref[...] = jnp.dot(x1, w_ref[...], preferred_element_type=jnp.float32)


def _mid(p, b, w):
    return pl.pallas_call(
        _mid_body,
        out_shape=jax.ShapeDtypeStruct((N, D), jnp.float32),
    )(p, b.reshape(1, D), w)


def _head_body(p_ref, b2_ref, batch_ref, wg1_ref, bg1_ref, wg2_ref, bg2_ref,
               wl1_ref, bl1_ref, wl2_ref, bl2_ref, o_ref):
    x2 = p_ref[0, :N] + p_ref[1, :N] + b2_ref[...]
    t = jnp.maximum(jnp.dot(x2, wg1_ref[...],
                            preferred_element_type=jnp.float32)
                    + bg1_ref[...], 0.0)
    g = jnp.dot(t, wg2_ref[...], preferred_element_type=jnp.float32) \
        + bg2_ref[...]

    gid = lax.broadcasted_iota(jnp.int32, (N, B), 1)
    onehot = batch_ref[...] == gid
    onehotf = onehot.astype(jnp.float32)

    m = jnp.max(jnp.where(onehot, g, -1e30), axis=0, keepdims=True)
    m_node = jnp.sum(onehotf * m, axis=1, keepdims=True)
    e = jnp.exp(g - m_node)
    denom = jnp.sum(onehotf * e, axis=0, keepdims=True)
    denom_node = jnp.sum(onehotf * denom, axis=1, keepdims=True)
    alpha = e / (denom_node + 1e-16)

    pooled = lax.dot_general(onehotf, alpha * x2, (((0,), (0,)), ((), ())),
                             preferred_element_type=jnp.float32)
    h = jnp.maximum(jnp.dot(pooled, wl1_ref[...],
                            preferred_element_type=jnp.float32)
                    + bl1_ref[...], 0.0)
    o_ref[...] = jnp.dot(h, wl2_ref[...],
                         preferred_element_type=jnp.float32) + bl2_ref[...]


def _head(p, b2, batch2d, Wg1, bg1, Wg2, bg2, Wl1, bl1, Wl2, bl2):
    return pl.pallas_call(
        _head_body,
        out_shape=jax.ShapeDtypeStruct((B, 1), jnp.float32),
    )(p, b2.reshape(1, D), batch2d, Wg1, bg1.reshape(1, D), Wg2,
      bg2.reshape(1, 1), Wl1, bl1.reshape(1, D), Wl2, bl2.reshape(1, 1))


def kernel(x, edge_index, edge_attr, batch, W1, b1, W2, b2,
           Wg1, bg1, Wg2, bg2, Wl1, bl1, Wl2, bl2):
    pad = NW * KCH * C - E
    pad_idx = jnp.arange(pad, dtype=jnp.int32) % N

    def lay(a, p):
        return jnp.concatenate([a, p]).reshape(NW, KCH, C)
    src3 = lay(edge_index[0], pad_idx)
    dst3 = lay(edge_index[1], pad_idx)
    w3 = lay(edge_attr, jnp.zeros((pad,), jnp.float32))
    zeros_nd = jnp.zeros((NPAD, D), jnp.float32)

    h1 = _mm(x, W1)
    p1 = _conv(h1, src3, dst3, w3, zeros_nd)
    h2 = _mid(p1, b1, W2)
    p2 = _conv(h2, src3, dst3, w3, zeros_nd)
    out = _head(p2, b2, batch.reshape(N, 1), Wg1, bg1, Wg2, bg2,
                Wl1, bl1, Wl2, bl2)
    return out[:, 0]


# speedup vs baseline: 5.5806x; 1.0709x over previous
import functools

import jax
import jax.numpy as jnp
from jax import lax
from jax.experimental import pallas as pl
from jax.experimental.pallas import tpu as pltpu
from jax.experimental.pallas import tpu_sc as plsc

N = 10000
E = 320000
D = 128
B = 64

NC = 2
NS = 16
NW = NC * NS
C = 128
KPH = 40
KCH = 80
KBUF = 40
KDATA = 80
NPAD = 10112
ROWS_PER_TILE = NPAD // NS


def _conv_body(h_hbm, src_hbm, dst_hbm, w_hbm, zero_hbm, out_hbm,
               srcb, dstb, wb, rows0, rows1, acc, gsem0, gsem1,
               ssem0, ssem1):
    c = lax.axis_index("c")
    s = lax.axis_index("s")
    wid = c * NS + s

    r0 = s * ROWS_PER_TILE
    pltpu.sync_copy(zero_hbm.at[pl.ds(r0, ROWS_PER_TILE)],
                    acc.at[pl.ds(r0, ROWS_PER_TILE)])

    def stage(phase0):
        pltpu.sync_copy(src_hbm.at[wid, pl.ds(phase0, KBUF)], srcb)
        pltpu.sync_copy(dst_hbm.at[wid, pl.ds(phase0, KBUF)], dstb)
        pltpu.sync_copy(w_hbm.at[wid, pl.ds(phase0, KBUF)], wb)

    def scale(rows, j):
        def grp_body(i16, carry2):
            w16 = wb[j, pl.ds(i16 * 16, 16)]
            for l in range(16):
                wl = w16[l]
                row = i16 * 16 + l
                for jj in range(D // 16):
                    sl = pl.ds(jj * 16, 16)
                    rows[row, sl] = rows[row, sl] * wl
            return carry2
        lax.fori_loop(0, C // 16, grp_body, 0)

    def phase():
        def pair(i, carry):
            j = 2 * i
            d0 = pltpu.async_copy(h_hbm.at[srcb.at[j]], rows0, gsem0)
            d1 = pltpu.async_copy(h_hbm.at[srcb.at[j + 1]], rows1, gsem1)
            d0.wait()
            scale(rows0, j)
            s0 = pltpu.async_copy(rows0, acc.at[dstb.at[j]], ssem0,
                                  add=True)
            d1.wait()
            scale(rows1, j + 1)
            s1 = pltpu.async_copy(rows1, acc.at[dstb.at[j + 1]], ssem1,
                                  add=True)
            s0.wait()
            s1.wait()
            return carry
        lax.fori_loop(0, KPH // 2, pair, 0)

    stage(0)
    plsc.subcore_barrier()
    phase()
    stage(KPH)
    phase()

    plsc.subcore_barrier()
    pltpu.sync_copy(acc.at[pl.ds(r0, ROWS_PER_TILE)],
                    out_hbm.at[c, pl.ds(r0, ROWS_PER_TILE)])


_conv = pl.kernel(
    _conv_body,
    out_type=jax.ShapeDtypeStruct((NC, NPAD, D), jnp.float32),
    mesh=plsc.VectorSubcoreMesh(core_axis_name="c", subcore_axis_name="s",
                                num_cores=NC, num_subcores=NS),
    scratch_types=[
        pltpu.VMEM((KBUF, C), jnp.int32),
        pltpu.VMEM((KBUF, C), jnp.int32),
        pltpu.VMEM((KBUF, C), jnp.float32),
        pltpu.VMEM((C, D), jnp.float32),
        pltpu.VMEM((C, D), jnp.float32),
        pltpu.VMEM_SHARED((NPAD, D), jnp.float32),
        pltpu.SemaphoreType.DMA,
        pltpu.SemaphoreType.DMA,
        pltpu.SemaphoreType.DMA,
        pltpu.SemaphoreType.DMA,
    ],
)


def _mm_body(x_ref, w_ref, o_ref):
    o_ref[...] = jnp.dot(x_ref[...], w_ref[...],
                         preferred_element_type=jnp.float32)


def _mm(x, w):
    return pl.pallas_call(
        _mm_body,
        out_shape=jax.ShapeDtypeStruct((x.shape[0], w.shape[1]), jnp.float32),
    )(x, w)


def _mid_body(p_ref, b_ref, w_ref, o_ref):
    x1 = jnp.maximum(p_ref[0, :N] + p_ref[1, :N] + b_ref[...], 0.0)
    o_ref[...] = jnp.dot(x1, w_ref[...], preferred_element_type=jnp.float32)


def _mid(p, b, w):
    return pl.pallas_call(
        _mid_body,
        out_shape=jax.ShapeDtypeStruct((N, D), jnp.float32),
    )(p, b.reshape(1, D), w)


def _head_body(p_ref, b2_ref, batch_ref, wg1_ref, bg1_ref, wg2_ref, bg2_ref,
               wl1_ref, bl1_ref, wl2_ref, bl2_ref, o_ref):
    x2 = p_ref[0, :N] + p_ref[1, :N] + b2_ref[...]
    t = jnp.maximum(jnp.dot(x2, wg1_ref[...],
                            preferred_element_type=jnp.float32)
                    + bg1_ref[...], 0.0)
    g = jnp.dot(t, wg2_ref[...], preferred_element_type=jnp.float32) \
        + bg2_ref[...]

    gid = lax.broadcasted_iota(jnp.int32, (N, B), 1)
    onehot = batch_ref[...] == gid
    onehotf = onehot.astype(jnp.float32)

    m = jnp.max(jnp.where(onehot, g, -1e30), axis=0, keepdims=True)
    m_node = jnp.sum(onehotf * m, axis=1, keepdims=True)
    e = jnp.exp(g - m_node)
    denom = jnp.sum(onehotf * e, axis=0, keepdims=True)
    denom_node = jnp.sum(onehotf * denom, axis=1, keepdims=True)
    alpha = e / (denom_node + 1e-16)

    pooled = lax.dot_general(onehotf, alpha * x2, (((0,), (0,)), ((), ())),
                             preferred_element_type=jnp.float32)
    h = jnp.maximum(jnp.dot(pooled, wl1_ref[...],
                            preferred_element_type=jnp.float32)
                    + bl1_ref[...], 0.0)
    o_ref[...] = jnp.dot(h, wl2_ref[...],
                         preferred_element_type=jnp.float32) + bl2_ref[...]


def _head(p, b2, batch2d, Wg1, bg1, Wg2, bg2, Wl1, bl1, Wl2, bl2):
    return pl.pallas_call(
        _head_body,
        out_shape=jax.ShapeDtypeStruct((B, 1), jnp.float32),
    )(p, b2.reshape(1, D), batch2d, Wg1, bg1.reshape(1, D), Wg2,
      bg2.reshape(1, 1), Wl1, bl1.reshape(1, D), Wl2, bl2.reshape(1, 1))


def kernel(x, edge_index, edge_attr, batch, W1, b1, W2, b2,
           Wg1, bg1, Wg2, bg2, Wl1, bl1, Wl2, bl2):
    pad = NW * KCH * C - E
    pad_idx = jnp.arange(pad, dtype=jnp.int32) % N

    def lay(a, p):
        return jnp.concatenate([a, p]).reshape(NW, KCH, C)
    src3 = lay(edge_index[0], pad_idx)
    dst3 = lay(edge_index[1], pad_idx)
    w3 = lay(edge_attr, jnp.zeros((pad,), jnp.float32))
    zeros_nd = jnp.zeros((NPAD, D), jnp.float32)

    h1 = _mm(x, W1)
    p1 = _conv(h1, src3, dst3, w3, zeros_nd)
    h2 = _mid(p1, b1, W2)
    p2 = _conv(h2, src3, dst3, w3, zeros_nd)
    out = _head(p2, b2, batch.reshape(N, 1), Wg1, bg1, Wg2, bg2,
                Wl1, bl1, Wl2, bl2)
    return out[:, 0]


# speedup vs baseline: 6.1124x; 1.0953x over previous
import functools

import jax
import jax.numpy as jnp
from jax import lax
from jax.experimental import pallas as pl
from jax.experimental.pallas import tpu as pltpu
from jax.experimental.pallas import tpu_sc as plsc

N = 10000
E = 320000
D = 128
B = 64

NC = 2
NS = 16
NW = NC * NS
C = 128
KPH = 40
KCH = 80
KBUF = 40
KDATA = 80
NPAD = 10112
ROWS_PER_TILE = NPAD // NS


def _conv_body(h_hbm, src_hbm, dst_hbm, w_hbm, zero_hbm, out_hbm,
               srcb, dstb, wb, rows0, rows1, acc, gsem0, gsem1,
               ssem0, ssem1):
    c = lax.axis_index("c")
    s = lax.axis_index("s")
    wid = c * NS + s

    r0 = s * ROWS_PER_TILE
    pltpu.sync_copy(zero_hbm.at[pl.ds(r0, ROWS_PER_TILE)],
                    acc.at[pl.ds(r0, ROWS_PER_TILE)])

    def stage(phase0):
        pltpu.sync_copy(src_hbm.at[wid, pl.ds(phase0, KBUF)], srcb)
        pltpu.sync_copy(dst_hbm.at[wid, pl.ds(phase0, KBUF)], dstb)
        pltpu.sync_copy(w_hbm.at[wid, pl.ds(phase0, KBUF)], wb)

    def scale(rows, j):
        def grp_body(i16, carry2):
            w16 = wb[j, pl.ds(i16 * 16, 16)]
            for l in range(16):
                wl = w16[l]
                row = i16 * 16 + l
                for jj in range(D // 16):
                    sl = pl.ds(jj * 16, 16)
                    rows[row, sl] = rows[row, sl] * wl
            return carry2
        lax.fori_loop(0, C // 16, grp_body, 0)

    def phase():
        def quad(i, carry):
            j = 4 * i
            d0 = pltpu.async_copy(h_hbm.at[srcb.at[j]], rows0, gsem0)
            d1 = pltpu.async_copy(h_hbm.at[srcb.at[j + 1]], rows1, gsem1)
            d0.wait()
            scale(rows0, j)
            s0 = pltpu.async_copy(rows0, acc.at[dstb.at[j]], ssem0,
                                  add=True)
            d1.wait()
            scale(rows1, j + 1)
            s1 = pltpu.async_copy(rows1, acc.at[dstb.at[j + 1]], ssem1,
                                  add=True)
            s0.wait()
            d2 = pltpu.async_copy(h_hbm.at[srcb.at[j + 2]], rows0, gsem0)
            s1.wait()
            d3 = pltpu.async_copy(h_hbm.at[srcb.at[j + 3]], rows1, gsem1)
            d2.wait()
            scale(rows0, j + 2)
            s2 = pltpu.async_copy(rows0, acc.at[dstb.at[j + 2]], ssem0,
                                  add=True)
            d3.wait()
            scale(rows1, j + 3)
            s3 = pltpu.async_copy(rows1, acc.at[dstb.at[j + 3]], ssem1,
                                  add=True)
            s2.wait()
            s3.wait()
            return carry
        lax.fori_loop(0, KPH // 4, quad, 0)

    stage(0)
    plsc.subcore_barrier()
    phase()
    stage(KPH)
    phase()

    plsc.subcore_barrier()
    pltpu.sync_copy(acc.at[pl.ds(r0, ROWS_PER_TILE)],
                    out_hbm.at[c, pl.ds(r0, ROWS_PER_TILE)])


_conv = pl.kernel(
    _conv_body,
    out_type=jax.ShapeDtypeStruct((NC, NPAD, D), jnp.float32),
    mesh=plsc.VectorSubcoreMesh(core_axis_name="c", subcore_axis_name="s",
                                num_cores=NC, num_subcores=NS),
    scratch_types=[
        pltpu.VMEM((KBUF, C), jnp.int32),
        pltpu.VMEM((KBUF, C), jnp.int32),
        pltpu.VMEM((KBUF, C), jnp.float32),
        pltpu.VMEM((C, D), jnp.float32),
        pltpu.VMEM((C, D), jnp.float32),
        pltpu.VMEM_SHARED((NPAD, D), jnp.float32),
        pltpu.SemaphoreType.DMA,
        pltpu.SemaphoreType.DMA,
        pltpu.SemaphoreType.DMA,
        pltpu.SemaphoreType.DMA,
    ],
)


def _mm_body(x_ref, w_ref, o_ref):
    o_ref[...] = jnp.dot(x_ref[...], w_ref[...],
                         preferred_element_type=jnp.float32)


def _mm(x, w):
    return pl.pallas_call(
        _mm_body,
        out_shape=jax.ShapeDtypeStruct((x.shape[0], w.shape[1]), jnp.float32),
    )(x, w)


def _mid_body(p_ref, b_ref, w_ref, o_ref):
    x1 = jnp.maximum(p_ref[0, :N] + p_ref[1, :N] + b_ref[...], 0.0)
    o_ref[...] = jnp.dot(x1, w_ref[...], preferred_element_type=jnp.float32)


def _mid(p, b, w):
    return pl.pallas_call(
        _mid_body,
        out_shape=jax.ShapeDtypeStruct((N, D), jnp.float32),
    )(p, b.reshape(1, D), w)


def _head_body(p_ref, b2_ref, batch_ref, wg1_ref, bg1_ref, wg2_ref, bg2_ref,
               wl1_ref, bl1_ref, wl2_ref, bl2_ref, o_ref):
    x2 = p_ref[0, :N] + p_ref[1, :N] + b2_ref[...]
    t = jnp.maximum(jnp.dot(x2, wg1_ref[...],
                            preferred_element_type=jnp.float32)
                    + bg1_ref[...], 0.0)
    g = jnp.dot(t, wg2_ref[...], preferred_element_type=jnp.float32) \
        + bg2_ref[...]

    gid = lax.broadcasted_iota(jnp.int32, (N, B), 1)
    onehot = batch_ref[...] == gid
    onehotf = onehot.astype(jnp.float32)

    m = jnp.max(jnp.where(onehot, g, -1e30), axis=0, keepdims=True)
    m_node = jnp.sum(onehotf * m, axis=1, keepdims=True)
    e = jnp.exp(g - m_node)
    denom = jnp.sum(onehotf * e, axis=0, keepdims=True)
    denom_node = jnp.sum(onehotf * denom, axis=1, keepdims=True)
    alpha = e / (denom_node + 1e-16)

    pooled = lax.dot_general(onehotf, alpha * x2, (((0,), (0,)), ((), ())),
                             preferred_element_type=jnp.float32)
    h = jnp.maximum(jnp.dot(pooled, wl1_ref[...],
                            preferred_element_type=jnp.float32)
                    + bl1_ref[...], 0.0)
    o_ref[...] = jnp.dot(h, wl2_ref[...],
                         preferred_element_type=jnp.float32) + bl2_ref[...]


def _head(p, b2, batch2d, Wg1, bg1, Wg2, bg2, Wl1, bl1, Wl2, bl2):
    return pl.pallas_call(
        _head_body,
        out_shape=jax.ShapeDtypeStruct((B, 1), jnp.float32),
    )(p, b2.reshape(1, D), batch2d, Wg1, bg1.reshape(1, D), Wg2,
      bg2.reshape(1, 1), Wl1, bl1.reshape(1, D), Wl2, bl2.reshape(1, 1))


def kernel(x, edge_index, edge_attr, batch, W1, b1, W2, b2,
           Wg1, bg1, Wg2, bg2, Wl1, bl1, Wl2, bl2):
    pad = NW * KCH * C - E
    pad_idx = jnp.arange(pad, dtype=jnp.int32) % N

    def lay(a, p):
        return jnp.concatenate([a, p]).reshape(NW, KCH, C)
    src3 = lay(edge_index[0], pad_idx)
    dst3 = lay(edge_index[1], pad_idx)
    w3 = lay(edge_attr, jnp.zeros((pad,), jnp.float32))
    zeros_nd = jnp.zeros((NPAD, D), jnp.float32)

    h1 = _mm(x, W1)
    p1 = _conv(h1, src3, dst3, w3, zeros_nd)
    h2 = _mid(p1, b1, W2)
    p2 = _conv(h2, src3, dst3, w3, zeros_nd)
    out = _head(p2, b2, batch.reshape(N, 1), Wg1, bg1, Wg2, bg2,
                Wl1, bl1, Wl2, bl2)
    return out[:, 0]
